# Initial kernel scaffold; baseline (speedup 1.0000x reference)
import jax
import jax.numpy as jnp
from jax.experimental import pallas as pl


def kernel(state_pred, hazard_logits, state_target, state_mask, event_times, event_indicators):
    raise NotImplementedError("write your pallas kernel here")


# speedup vs baseline: 5.0099x; 5.0099x over previous
import functools

import jax
import jax.numpy as jnp
from jax import lax
from jax.experimental import pallas as pl
from jax.experimental.pallas import tpu as pltpu

NUM_EVENTS = 5
NUM_INTERVALS = 20
BATCH = 16384
NUM_TARGETS = 128
STATE_WEIGHT = 1.0
SURVIVAL_WEIGHT = 1.0

GRID = 16
ROWS_BLK = BATCH // GRID
FLAT = NUM_EVENTS * BATCH
COLS_BLK = FLAT // GRID


def _tc_body(sp_ref, st_ref, sm_ref, hz_ref, t_ref, ind_ref, out_ref, acc_ref):
    i = pl.program_id(0)

    @pl.when(i == 0)
    def _init():
        acc_ref[0] = 0.0
        acc_ref[1] = 0.0
        acc_ref[2] = 0.0

    sp = sp_ref[...]
    st = st_ref[...]
    sm = sm_ref[...]
    d = sp - st
    mse_part = jnp.sum(d * d * sm)
    msum_part = jnp.sum(sm)

    x = hz_ref[...]
    tt = t_ref[...].reshape(1, COLS_BLK)
    ind = ind_ref[...].reshape(1, COLS_BLK)

    jj = lax.broadcasted_iota(jnp.int32, (NUM_INTERVALS, COLS_BLK), 0)
    bj = (jj.astype(jnp.float32) + 1.0) * 0.5
    cmp = tt > bj

    ex = jnp.exp(-x)
    p = 1.0 / (1.0 + ex)
    l1mp = jnp.log((1.0 - p) + 1e-8)
    mask_a = jnp.logical_and(cmp, jj < NUM_INTERVALS - 1)
    s_surv = jnp.sum(jnp.where(mask_a, l1mp, 0.0))

    idx = jnp.sum(cmp.astype(jnp.int32), axis=0, keepdims=True)
    idxc = jnp.minimum(idx, NUM_INTERVALS - 1)
    sel_b = jj == idxc
    xg = jnp.sum(jnp.where(sel_b, x, 0.0), axis=0, keepdims=True)
    pg = 1.0 / (1.0 + jnp.exp(-xg))
    lp = jnp.log(pg + 1e-8)
    s_haz = jnp.sum(jnp.where(ind > 0.5, lp, 0.0))

    acc_ref[0] = acc_ref[0] + mse_part
    acc_ref[1] = acc_ref[1] + msum_part
    acc_ref[2] = acc_ref[2] + (s_surv + s_haz)

    @pl.when(i == GRID - 1)
    def _fin():
        state_loss = acc_ref[0] / (acc_ref[1] + 1e-8)
        surv_loss = -acc_ref[2] / jnp.float32(FLAT)
        out_ref[0, 0] = STATE_WEIGHT * state_loss + SURVIVAL_WEIGHT * surv_loss


@functools.partial(jax.jit)
def kernel(state_pred, hazard_logits, state_target, state_mask,
           event_times, event_indicators):
    hz_t = jnp.transpose(hazard_logits, (2, 0, 1)).reshape(NUM_INTERVALS, FLAT)
    t_t = jnp.transpose(event_times, (1, 0)).reshape(GRID, 1, COLS_BLK)
    ind_t = jnp.transpose(event_indicators, (1, 0)).reshape(GRID, 1, COLS_BLK)

    out = pl.pallas_call(
        _tc_body,
        grid=(GRID,),
        in_specs=[
            pl.BlockSpec((ROWS_BLK, NUM_TARGETS), lambda i: (i, 0)),
            pl.BlockSpec((ROWS_BLK, NUM_TARGETS), lambda i: (i, 0)),
            pl.BlockSpec((ROWS_BLK, NUM_TARGETS), lambda i: (i, 0)),
            pl.BlockSpec((NUM_INTERVALS, COLS_BLK), lambda i: (0, i)),
            pl.BlockSpec((1, 1, COLS_BLK), lambda i: (i, 0, 0)),
            pl.BlockSpec((1, 1, COLS_BLK), lambda i: (i, 0, 0)),
        ],
        out_specs=pl.BlockSpec(memory_space=pltpu.SMEM),
        out_shape=jax.ShapeDtypeStruct((1, 1), jnp.float32),
        scratch_shapes=[pltpu.SMEM((4,), jnp.float32)],
    )(state_pred, state_target, state_mask, hz_t, t_t, ind_t)
    return out[0, 0]
